# baseline (speedup 1.0000x reference)
import functools

import jax
import jax.numpy as jnp
from jax import lax
from jax.experimental import pallas as pl
from jax.experimental.pallas import tpu as pltpu
from jax.experimental.pallas import tpu_sc as plsc

_B, _F, _G, _S = 16384, 208, 26, 8
_NC, _NS, _L = 2, 16, 16
_NJ = _F // _L
_GPAD = 32
_RB = 1024


def _sc_weights_body(gidx_hbm, wg_hbm, wfc_hbm, w_hbm, gidx_v, wg_v, wfc_v, w_v):
    wid = lax.axis_index("s") * _NC + lax.axis_index("c")

    pltpu.sync_copy(gidx_hbm, gidx_v)
    pltpu.sync_copy(wg_hbm, wg_v)
    pltpu.sync_copy(wfc_hbm, wfc_v)

    for j in range(_NJ):
        w_v[pl.ds(j * _L, _L)] = jnp.zeros((_L,), jnp.float32)

    lane_lo = lax.iota(jnp.int32, _L) < _S
    wfc_a = wfc_v[pl.ds(0, _L)]
    wfc_b = wfc_v[pl.ds(_L, _L)]

    def _wfc_at(g):
        return wfc_a[g] if g < _L else wfc_b[g - _L]

    for j in range(_NJ):
        wfc_g = jnp.where(lane_lo,
                          jnp.full((_L,), _wfc_at(2 * j), jnp.float32),
                          jnp.full((_L,), _wfc_at(2 * j + 1), jnp.float32))
        prod = wg_v[pl.ds(j * _L, _L)] * wfc_g
        plsc.addupdate_scatter(w_v, [gidx_v[pl.ds(j * _L, _L)]], prod)

    @pl.when(wid == 0)
    def _():
        pltpu.sync_copy(w_v, w_hbm)


def _sc_weights(gidx, wg, wfc):
    mesh = plsc.VectorSubcoreMesh(core_axis_name="c", subcore_axis_name="s")
    return pl.kernel(
        _sc_weights_body,
        out_type=jax.ShapeDtypeStruct((_F,), jnp.float32),
        mesh=mesh,
        scratch_types=[
            pltpu.VMEM((_F,), jnp.int32),
            pltpu.VMEM((_F,), jnp.float32),
            pltpu.VMEM((_GPAD,), jnp.float32),
            pltpu.VMEM((_F,), jnp.float32),
        ],
        compiler_params=pltpu.CompilerParams(needs_layout_passes=False),
    )(gidx, wg, wfc)


def _tc_matvec_body(w_ref, x_ref, out_ref):
    out_ref[...] = jnp.dot(x_ref[...], w_ref[...],
                           preferred_element_type=jnp.float32)


@jax.jit
def _tc_matvec(x, w_col):
    grid = _B // _RB
    return pl.pallas_call(
        _tc_matvec_body,
        grid=(grid,),
        in_specs=[
            pl.BlockSpec((_F, 1), lambda i: (0, 0)),
            pl.BlockSpec((_RB, _F), lambda i: (i, 0)),
        ],
        out_specs=pl.BlockSpec((_RB, 1), lambda i: (i, 0)),
        out_shape=jax.ShapeDtypeStruct((_B, 1), jnp.float32),
        compiler_params=pltpu.CompilerParams(
            dimension_semantics=("arbitrary",)),
    )(w_col, x)


def kernel(x, group_idx, W_g, W_fc):
    gidx = group_idx.reshape(_F).astype(jnp.int32)
    wg = W_g.reshape(_F)
    wfc = jnp.pad(W_fc.reshape(_G), (0, _GPAD - _G))
    w = _sc_weights(gidx, wg, wfc)
    return _tc_matvec(x, w.reshape(_F, 1))


# speedup vs baseline: 1.9253x; 1.9253x over previous
import jax
import jax.numpy as jnp
from jax import lax
from jax.experimental import pallas as pl
from jax.experimental.pallas import tpu as pltpu
from jax.experimental.pallas import tpu_sc as plsc

_B, _F, _G, _S = 16384, 208, 26, 8
_NC, _NS, _L = 2, 16, 16
_NJ = _F // _L
_GPAD = 32
_PACK = _F + _GPAD + _F
_BC = 2048


def _sc_weights_body(pack_hbm, w_hbm, pack_v, w_v):
    wid = lax.axis_index("s") * _NC + lax.axis_index("c")

    pltpu.sync_copy(pack_hbm, pack_v)

    for j in range(_NJ):
        w_v[pl.ds(j * _L, _L)] = jnp.zeros((_L,), jnp.float32)

    lane_lo = lax.iota(jnp.int32, _L) < _S
    wfc_a = plsc.bitcast(pack_v[pl.ds(_F, _L)], jnp.float32)
    wfc_b = plsc.bitcast(pack_v[pl.ds(_F + _L, _L)], jnp.float32)

    def _wfc_at(g):
        return wfc_a[g] if g < _L else wfc_b[g - _L]

    for j in range(_NJ):
        wfc_g = jnp.where(lane_lo,
                          jnp.full((_L,), _wfc_at(2 * j), jnp.float32),
                          jnp.full((_L,), _wfc_at(2 * j + 1), jnp.float32))
        wg = plsc.bitcast(pack_v[pl.ds(j * _L, _L)], jnp.float32)
        prod = wg * wfc_g
        gidx = pack_v[pl.ds(_F + _GPAD + j * _L, _L)]
        plsc.addupdate_scatter(w_v, [gidx], prod)

    @pl.when(wid == 0)
    def _():
        pltpu.sync_copy(w_v, w_hbm)


def _sc_weights(pack):
    mesh = plsc.VectorSubcoreMesh(core_axis_name="c", subcore_axis_name="s")
    return pl.kernel(
        _sc_weights_body,
        out_type=jax.ShapeDtypeStruct((_F,), jnp.float32),
        mesh=mesh,
        scratch_types=[
            pltpu.VMEM((_PACK,), jnp.int32),
            pltpu.VMEM((_F,), jnp.float32),
        ],
        compiler_params=pltpu.CompilerParams(needs_layout_passes=False),
    )(pack)


def _tc_matvec_body(w_ref, xT_ref, out_ref):
    out_ref[...] = jnp.sum(xT_ref[...] * w_ref[...], axis=0)


def _tc_matvec(xT, w_col):
    grid = _B // _BC
    return pl.pallas_call(
        _tc_matvec_body,
        grid=(grid,),
        in_specs=[
            pl.BlockSpec((_F, 1), lambda i: (0, 0)),
            pl.BlockSpec((_F, _BC), lambda i: (0, i)),
        ],
        out_specs=pl.BlockSpec((_BC,), lambda i: (i,)),
        out_shape=jax.ShapeDtypeStruct((_B,), jnp.float32),
        compiler_params=pltpu.CompilerParams(
            dimension_semantics=("arbitrary",)),
    )(w_col, xT)


def kernel(x, group_idx, W_g, W_fc):
    wfc = jnp.pad(W_fc.reshape(_G), (0, _GPAD - _G))
    wg_bits = lax.bitcast_convert_type(W_g.reshape(_F), jnp.int32)
    wfc_bits = lax.bitcast_convert_type(wfc, jnp.int32)
    pack = jnp.concatenate([wg_bits, wfc_bits,
                            group_idx.reshape(_F).astype(jnp.int32)])
    w = _sc_weights(pack)
    out = _tc_matvec(x.T, w.reshape(_F, 1))
    return out.reshape(_B, 1)


# speedup vs baseline: 2.0551x; 1.0674x over previous
import jax
import jax.numpy as jnp
from jax import lax
from jax.experimental import pallas as pl
from jax.experimental.pallas import tpu as pltpu
from jax.experimental.pallas import tpu_sc as plsc

_B, _F, _G, _S = 16384, 208, 26, 8
_NC, _NS, _L = 2, 16, 16
_NJ = _F // _L
_GPAD = 32
_PACK = _F + _GPAD + _F
_BC = 4096


def _sc_weights_body(pack_hbm, w_hbm, pack_v, w_v):
    wid = lax.axis_index("s") * _NC + lax.axis_index("c")

    pltpu.sync_copy(pack_hbm, pack_v)

    zeros = jnp.zeros((_L,), jnp.float32)
    lanes = lax.iota(jnp.int32, _L)

    def _zero_body(j, carry):
        w_v[pl.ds(j * _L, _L)] = zeros
        return carry

    lax.fori_loop(0, _NJ, _zero_body, 0)

    def _chunk_body(j, carry):
        p = lanes + j * _L
        g_ids = lax.shift_right_logical(p, jnp.int32(3))
        wfc_g = plsc.bitcast(
            plsc.load_gather(pack_v, [g_ids + jnp.int32(_F)]), jnp.float32)
        wg = plsc.bitcast(plsc.load_gather(pack_v, [p]), jnp.float32)
        gidx = plsc.load_gather(pack_v, [p + jnp.int32(_F + _GPAD)])
        plsc.addupdate_scatter(w_v, [gidx], wg * wfc_g)
        return carry

    lax.fori_loop(0, _NJ, _chunk_body, 0)

    @pl.when(wid == 0)
    def _():
        pltpu.sync_copy(w_v, w_hbm)


def _sc_weights(pack):
    mesh = plsc.VectorSubcoreMesh(core_axis_name="c", subcore_axis_name="s")
    return pl.kernel(
        _sc_weights_body,
        out_type=jax.ShapeDtypeStruct((_F,), jnp.float32),
        mesh=mesh,
        scratch_types=[
            pltpu.VMEM((_PACK,), jnp.int32),
            pltpu.VMEM((_F,), jnp.float32),
        ],
        compiler_params=pltpu.CompilerParams(needs_layout_passes=False),
    )(pack)


def _tc_matvec_body(w_ref, xT_ref, out_ref):
    out_ref[...] = jnp.sum(xT_ref[...] * w_ref[...], axis=0)


def _tc_matvec(xT, w_col):
    grid = _B // _BC
    return pl.pallas_call(
        _tc_matvec_body,
        grid=(grid,),
        in_specs=[
            pl.BlockSpec((_F, 1), lambda i: (0, 0)),
            pl.BlockSpec((_F, _BC), lambda i: (0, i)),
        ],
        out_specs=pl.BlockSpec((_BC,), lambda i: (i,)),
        out_shape=jax.ShapeDtypeStruct((_B,), jnp.float32),
        compiler_params=pltpu.CompilerParams(
            dimension_semantics=("arbitrary",)),
    )(w_col, xT)


def kernel(x, group_idx, W_g, W_fc):
    wfc = jnp.pad(W_fc.reshape(_G), (0, _GPAD - _G))
    wg_bits = lax.bitcast_convert_type(W_g.reshape(_F), jnp.int32)
    wfc_bits = lax.bitcast_convert_type(wfc, jnp.int32)
    pack = jnp.concatenate([wg_bits, wfc_bits,
                            group_idx.reshape(_F).astype(jnp.int32)])
    w = _sc_weights(pack)
    out = _tc_matvec(x.T, w.reshape(_F, 1))
    return out.reshape(_B, 1)


# speedup vs baseline: 5.3872x; 2.6214x over previous
import jax
import jax.numpy as jnp
from jax import lax
from jax.experimental import pallas as pl
from jax.experimental.pallas import tpu as pltpu

_B, _F, _G, _S = 16384, 208, 26, 8
_BC = 4096


def _tc_body(wg_ref, wfc_ref, x3_ref, out_ref):
    w3 = wg_ref[...] * wfc_ref[...]
    t = jnp.sum(x3_ref[...] * w3, axis=1)
    out_ref[...] = jnp.sum(t, axis=0)


def _tc_matvec(x3, W_g, wfc3):
    grid = _B // _BC
    return pl.pallas_call(
        _tc_body,
        grid=(grid,),
        in_specs=[
            pl.BlockSpec((_G, _S, 1), lambda i: (0, 0, 0)),
            pl.BlockSpec((_G, 1, 1), lambda i: (0, 0, 0)),
            pl.BlockSpec((_G, _S, _BC), lambda i: (0, 0, i)),
        ],
        out_specs=pl.BlockSpec((_BC,), lambda i: (i,)),
        out_shape=jax.ShapeDtypeStruct((_B,), jnp.float32),
        compiler_params=pltpu.CompilerParams(
            dimension_semantics=("arbitrary",)),
    )(W_g, wfc3, x3)


def kernel(x, group_idx, W_g, W_fc):
    x3 = x.T.reshape(_G, _S, _B)
    out = _tc_matvec(x3, W_g, W_fc.reshape(_G, 1, 1))
    return out.reshape(_B, 1)
